# Initial kernel scaffold; baseline (speedup 1.0000x reference)
import jax
import jax.numpy as jnp
from jax.experimental import pallas as pl


def kernel(x, edge_index, W_qkv, b_qkv, W_o, b_o, W1, b1, W2, b2):
    raise NotImplementedError("write your pallas kernel here")


# speedup vs baseline: 11.1910x; 11.1910x over previous
import functools

import jax
import jax.numpy as jnp
import numpy as np
from jax import lax
from jax.experimental import pallas as pl
from jax.experimental.pallas import tpu as pltpu
from jax.experimental.pallas import tpu_sc as plsc

N = 10000
E = 320000
DIM = 128
H = 8
HD = 16

NC = 2
NS = 16
HC = H // NC
CD = HC * HD
EPS = E // NS
CH = 80
NCHUNK = EPS // CH
NP = 10240
RPW = NP // NS
WB = 128


_BLK = 2000


def _qkv_body(x_ref, w_ref, bq_ref, bk_ref, bv_ref, q_ref, k_ref, v_ref):
    acc = jnp.dot(x_ref[...], w_ref[...], preferred_element_type=jnp.float32,
                  precision=lax.Precision.HIGHEST)
    q = acc[:, :DIM] + bq_ref[...]
    k = acc[:, DIM:2 * DIM] + bk_ref[...]
    v = acc[:, 2 * DIM:] + bv_ref[...]
    q_ref[0] = q[:, :CD]
    q_ref[1] = q[:, CD:]
    k_ref[0] = k[:, :CD]
    k_ref[1] = k[:, CD:]
    v_ref[0] = v[:, :CD]
    v_ref[1] = v[:, CD:]


def _qkv_project(x, w_perm, bq, bk, bv):
    grid = N // _BLK
    return pl.pallas_call(
        _qkv_body,
        grid=(grid,),
        in_specs=[
            pl.BlockSpec((_BLK, DIM), lambda i: (i, 0)),
            pl.BlockSpec((DIM, 3 * DIM), lambda i: (0, 0)),
            pl.BlockSpec((1, DIM), lambda i: (0, 0)),
            pl.BlockSpec((1, DIM), lambda i: (0, 0)),
            pl.BlockSpec((1, DIM), lambda i: (0, 0)),
        ],
        out_specs=[
            pl.BlockSpec((NC, _BLK, CD), lambda i: (0, i, 0)),
            pl.BlockSpec((NC, _BLK, CD), lambda i: (0, i, 0)),
            pl.BlockSpec((NC, _BLK, CD), lambda i: (0, i, 0)),
        ],
        out_shape=[jax.ShapeDtypeStruct((NC, N, CD), jnp.float32)] * 3,
    )(x, w_perm, bq, bk, bv)


def _edge_body(q_hbm, k_hbm, v_hbm, src_hbm, dst_hbm, zagg_hbm, zsum_hbm,
               agg_out, sum_out, src_v, dst_v, dstq_v, krows, qrows, vrows,
               exb, sem, stg, stg8, agg_sh, sum_sh):
    cid = lax.axis_index("c")
    sid = lax.axis_index("s")

    for i in range(RPW // WB):
        off = sid * RPW + i * WB
        pltpu.sync_copy(zagg_hbm.at[pl.ds(i * WB, WB)], stg)
        pltpu.sync_copy(stg, agg_sh.at[pl.ds(off, WB)])
    pltpu.sync_copy(zsum_hbm.at[pl.ds(0, RPW)], stg8)
    pltpu.sync_copy(stg8, sum_sh.at[pl.ds(sid * RPW, RPW)])
    plsc.subcore_barrier()

    lanes = lax.iota(jnp.int32, 16)
    scale = jnp.float32(1.0 / np.sqrt(HD))
    roff = jnp.broadcast_to(cid * N, (16,)).astype(jnp.int32)

    zero16 = jnp.zeros((16,), jnp.float32)
    for g in range(CH // 16):
        e16 = g * 16 + lanes
        for h in range(HC, H):
            plsc.store_scatter(exb, [e16, jnp.full((16,), h, jnp.int32)],
                               zero16)

    def chunk_body(c, _):
        base = sid * EPS + c * CH
        pltpu.sync_copy(src_hbm.at[pl.ds(base, CH)], src_v)
        pltpu.sync_copy(dst_hbm.at[pl.ds(base, CH)], dst_v)
        for g in range(CH // 16):
            sl = pl.ds(g * 16, 16)
            src_v[sl] = src_v[sl] + roff
            dstq_v[sl] = dst_v[sl] + roff
        pltpu.async_copy(k_hbm.at[src_v], krows, sem).wait()
        pltpu.async_copy(q_hbm.at[dstq_v], qrows, sem).wait()
        pltpu.async_copy(v_hbm.at[src_v], vrows, sem).wait()

        def group_body(g, _):
            e16 = g * 16 + lanes
            for h in range(HC):
                acc = jnp.zeros((16,), jnp.float32)
                for d in range(HD):
                    col = jnp.full((16,), h * HD + d, jnp.int32)
                    kv = plsc.load_gather(krows, [e16, col])
                    qv = plsc.load_gather(qrows, [e16, col])
                    acc = acc + kv * qv
                ex = jnp.exp(acc * scale)
                plsc.store_scatter(exb, [e16, jnp.full((16,), h, jnp.int32)],
                                   ex)
                for d in range(HD):
                    col = jnp.full((16,), h * HD + d, jnp.int32)
                    vv = plsc.load_gather(vrows, [e16, col])
                    plsc.store_scatter(vrows, [e16, col], vv * ex)
            return 0

        lax.fori_loop(0, CH // 16, group_body, 0)

        pltpu.sync_copy(exb, sum_sh.at[dst_v], add=True)
        pltpu.sync_copy(vrows, agg_sh.at[dst_v], add=True)
        return 0

    lax.fori_loop(0, NCHUNK, chunk_body, 0)
    plsc.subcore_barrier()

    for i in range(RPW // WB):
        off = sid * RPW + i * WB
        pltpu.sync_copy(agg_sh.at[pl.ds(off, WB)], stg)
        pltpu.sync_copy(stg, agg_out.at[cid, pl.ds(off, WB)])
    pltpu.sync_copy(sum_sh.at[pl.ds(sid * RPW, RPW)], stg8)
    pltpu.sync_copy(stg8, sum_out.at[cid, pl.ds(sid * RPW, RPW)])


@functools.cache
def _edge_kernel_fn():
    return functools.partial(
        pl.kernel,
        out_type=[
            jax.ShapeDtypeStruct((NC, NP, CD), jnp.float32),
            jax.ShapeDtypeStruct((NC, NP, H), jnp.float32),
        ],
        mesh=plsc.VectorSubcoreMesh(core_axis_name="c", subcore_axis_name="s",
                                    num_cores=NC, num_subcores=NS),
        compiler_params=pltpu.CompilerParams(needs_layout_passes=False,
                                             use_tc_tiling_on_sc=False),
        scratch_types=[
            pltpu.VMEM((CH,), jnp.int32),
            pltpu.VMEM((CH,), jnp.int32),
            pltpu.VMEM((CH,), jnp.int32),
            pltpu.VMEM((CH, CD), jnp.float32),
            pltpu.VMEM((CH, CD), jnp.float32),
            pltpu.VMEM((CH, CD), jnp.float32),
            pltpu.VMEM((CH, H), jnp.float32),
            pltpu.SemaphoreType.DMA,
            pltpu.VMEM((WB, CD), jnp.float32),
            pltpu.VMEM((RPW, H), jnp.float32),
            pltpu.VMEM_SHARED((NP, CD), jnp.float32),
            pltpu.VMEM_SHARED((NP, H), jnp.float32),
        ],
    )(_edge_body)


def _epi_body(x_ref, agg_ref, sum_ref, wo_ref, bo_ref, w1a_ref, w1b_ref,
              b1_ref, w2_ref, b2_ref, o_ref):
    ssum = jnp.concatenate([sum_ref[0], sum_ref[1]], axis=1)
    rec = jnp.where(ssum > 0, 1.0 / ssum, 0.0)
    rows = lax.broadcasted_iota(jnp.int32, (2 * H, DIM), 0)
    cols = lax.broadcasted_iota(jnp.int32, (2 * H, DIM), 1)
    hit = (rows // H == cols // CD) & (rows % H == (cols % CD) // HD)
    expand = jnp.where(hit, 1.0, 0.0).astype(jnp.float32)
    recx = jnp.dot(rec, expand, preferred_element_type=jnp.float32,
                   precision=lax.Precision.HIGHEST)
    agg = jnp.concatenate([agg_ref[0], agg_ref[1]], axis=1) * recx
    aggo = jnp.dot(agg, wo_ref[...], preferred_element_type=jnp.float32,
                   precision=lax.Precision.HIGHEST) + bo_ref[...]
    f = (jnp.dot(x_ref[...], w1a_ref[...], preferred_element_type=jnp.float32,
                 precision=lax.Precision.HIGHEST)
         + jnp.dot(aggo, w1b_ref[...], preferred_element_type=jnp.float32,
                   precision=lax.Precision.HIGHEST) + b1_ref[...])
    g = 0.5 * f * (1.0 + lax.erf(f * jnp.float32(1.0 / np.sqrt(2.0))))
    o_ref[...] = jnp.dot(g, w2_ref[...], preferred_element_type=jnp.float32,
                         precision=lax.Precision.HIGHEST) + b2_ref[...]


def _epilogue(x, agg_p, sum_p, W_o, b_o, W1a, W1b, b1, W2, b2):
    grid = N // _BLK
    return pl.pallas_call(
        _epi_body,
        grid=(grid,),
        in_specs=[
            pl.BlockSpec((_BLK, DIM), lambda i: (i, 0)),
            pl.BlockSpec((NC, _BLK, CD), lambda i: (0, i, 0)),
            pl.BlockSpec((NC, _BLK, H), lambda i: (0, i, 0)),
            pl.BlockSpec((DIM, DIM), lambda i: (0, 0)),
            pl.BlockSpec((1, DIM), lambda i: (0, 0)),
            pl.BlockSpec((DIM, DIM), lambda i: (0, 0)),
            pl.BlockSpec((DIM, DIM), lambda i: (0, 0)),
            pl.BlockSpec((1, DIM), lambda i: (0, 0)),
            pl.BlockSpec((DIM, DIM), lambda i: (0, 0)),
            pl.BlockSpec((1, DIM), lambda i: (0, 0)),
        ],
        out_specs=pl.BlockSpec((_BLK, DIM), lambda i: (i, 0)),
        out_shape=jax.ShapeDtypeStruct((N, DIM), jnp.float32),
    )(x, agg_p, sum_p, W_o, b_o, W1a, W1b, b1, W2, b2)


_PERM = np.concatenate([
    np.concatenate([np.arange(h * 3 * HD + j * HD, h * 3 * HD + (j + 1) * HD)
                    for h in range(H)])
    for j in range(3)
])


def kernel(x, edge_index, W_qkv, b_qkv, W_o, b_o, W1, b1, W2, b2):
    w_perm = W_qkv[:, _PERM]
    b_perm = b_qkv[_PERM]
    bq = b_perm[:DIM].reshape(1, DIM)
    bk = b_perm[DIM:2 * DIM].reshape(1, DIM)
    bv = b_perm[2 * DIM:].reshape(1, DIM)

    q, k, v = _qkv_project(x, w_perm, bq, bk, bv)

    src = edge_index[0]
    dst = edge_index[1]
    zagg = jnp.zeros((WB * (RPW // WB), CD), jnp.float32)
    zsum = jnp.zeros((RPW, H), jnp.float32)
    agg_p, sum_p = _edge_kernel_fn()(
        q.reshape(NC * N, CD), k.reshape(NC * N, CD), v.reshape(NC * N, CD),
        src, dst, zagg, zsum)

    return _epilogue(x, agg_p, sum_p, W_o, b_o.reshape(1, DIM),
                     W1[:DIM], W1[DIM:], b1.reshape(1, DIM), W2,
                     b2.reshape(1, DIM))


# speedup vs baseline: 13.2623x; 1.1851x over previous
import functools

import jax
import jax.numpy as jnp
import numpy as np
from jax import lax
from jax.experimental import pallas as pl
from jax.experimental.pallas import tpu as pltpu
from jax.experimental.pallas import tpu_sc as plsc

N = 10000
E = 320000
DIM = 128
H = 8
HD = 16

NC = 2
NS = 16
HC = H // NC
CD = HC * HD
EPS = E // NS
CH = 80
NCHUNK = EPS // CH
NP = 10112
RPW = NP // NS
WBP = (128, 128, 128, 128, 120)
WB = 128


_BLK = 2000


def _qkv_body(x_ref, w_ref, bq_ref, bk_ref, bv_ref, q_ref, k_ref, v_ref):
    acc = jnp.dot(x_ref[...], w_ref[...], preferred_element_type=jnp.float32,
                  precision=lax.Precision.HIGHEST)
    q = acc[:, :DIM] + bq_ref[...]
    k = acc[:, DIM:2 * DIM] + bk_ref[...]
    v = acc[:, 2 * DIM:] + bv_ref[...]
    q_ref[0] = q[:, :CD]
    q_ref[1] = q[:, CD:]
    k_ref[0] = k[:, :CD]
    k_ref[1] = k[:, CD:]
    v_ref[0] = v[:, :CD]
    v_ref[1] = v[:, CD:]


def _qkv_project(x, w_perm, bq, bk, bv):
    grid = N // _BLK
    return pl.pallas_call(
        _qkv_body,
        grid=(grid,),
        in_specs=[
            pl.BlockSpec((_BLK, DIM), lambda i: (i, 0)),
            pl.BlockSpec((DIM, 3 * DIM), lambda i: (0, 0)),
            pl.BlockSpec((1, DIM), lambda i: (0, 0)),
            pl.BlockSpec((1, DIM), lambda i: (0, 0)),
            pl.BlockSpec((1, DIM), lambda i: (0, 0)),
        ],
        out_specs=[
            pl.BlockSpec((NC, _BLK, CD), lambda i: (0, i, 0)),
            pl.BlockSpec((NC, _BLK, CD), lambda i: (0, i, 0)),
            pl.BlockSpec((NC, _BLK, CD), lambda i: (0, i, 0)),
        ],
        out_shape=[jax.ShapeDtypeStruct((NC, N, CD), jnp.float32)] * 3,
    )(x, w_perm, bq, bk, bv)


def _edge_body(q_hbm, k_hbm, v_hbm, src_hbm, dst_hbm, zagg_hbm, zsum_hbm,
               agg_out, sum_out, src_all, dst_all, dst_s, dstq_s, krows,
               qrows, vrows, exb, gsem, ssem, stg, stg8, agg_sh, sum_sh):
    cid = lax.axis_index("c")
    sid = lax.axis_index("s")

    poff = 0
    for w in WBP:
        off = sid * RPW + poff
        pltpu.sync_copy(zagg_hbm.at[pl.ds(poff, w)], stg.at[pl.ds(0, w)])
        pltpu.sync_copy(stg.at[pl.ds(0, w)], agg_sh.at[pl.ds(off, w)])
        poff += w
    pltpu.sync_copy(zsum_hbm.at[pl.ds(0, RPW)], stg8)
    pltpu.sync_copy(stg8, sum_sh.at[pl.ds(sid * RPW, RPW)])
    plsc.subcore_barrier()

    lanes = lax.iota(jnp.int32, 16)
    scale = jnp.float32(1.0 / np.sqrt(HD))
    roff = jnp.broadcast_to(cid * N, (16,)).astype(jnp.int32)

    pltpu.sync_copy(src_hbm.at[pl.ds(sid * EPS, EPS)], src_all)
    pltpu.sync_copy(dst_hbm.at[pl.ds(sid * EPS, EPS)], dst_all)

    def adj_body(i, _):
        sl = pl.ds(i * 16, 16)
        src_all[sl] = src_all[sl] + roff
        return 0

    lax.fori_loop(0, EPS // 16, adj_body, 0)

    zero16 = jnp.zeros((16,), jnp.float32)
    for b in range(2):
        for g in range(CH // 16):
            e16 = g * 16 + lanes
            for h in range(HC, H):
                plsc.store_scatter(exb[b], [e16, jnp.full((16,), h, jnp.int32)],
                                   zero16)

    def prep_indices(c, b):
        for g in range(CH // 16):
            sl = pl.ds(g * 16, 16)
            d16 = dst_all[pl.ds(c * CH + g * 16, 16)]
            dst_s[b][sl] = d16
            dstq_s[b][sl] = d16 + roff

    def issue_gathers(c, b):
        base = pl.ds(c * CH, CH)
        pltpu.async_copy(k_hbm.at[src_all.at[base]], krows[b], gsem[b][0])
        pltpu.async_copy(q_hbm.at[dstq_s[b]], qrows[b], gsem[b][1])
        pltpu.async_copy(v_hbm.at[src_all.at[base]], vrows[b], gsem[b][2])

    def wait_gathers(c, b):
        base = pl.ds(c * CH, CH)
        pltpu.make_async_copy(k_hbm.at[src_all.at[base]], krows[b],
                              gsem[b][0]).wait()
        pltpu.make_async_copy(q_hbm.at[dstq_s[b]], qrows[b], gsem[b][1]).wait()
        pltpu.make_async_copy(v_hbm.at[src_all.at[base]], vrows[b],
                              gsem[b][2]).wait()

    def issue_scatters(b):
        pltpu.async_copy(exb[b], sum_sh.at[dst_s[b]], ssem[b][0], add=True)
        pltpu.async_copy(vrows[b], agg_sh.at[dst_s[b]], ssem[b][1], add=True)

    def wait_scatters(b):
        pltpu.make_async_copy(exb[b], sum_sh.at[dst_s[b]], ssem[b][0]).wait()
        pltpu.make_async_copy(vrows[b], agg_sh.at[dst_s[b]], ssem[b][1]).wait()

    def compute(b):
        def group_body(g, _):
            e16 = g * 16 + lanes
            for h in range(HC):
                acc = jnp.zeros((16,), jnp.float32)
                for d in range(HD):
                    col = jnp.full((16,), h * HD + d, jnp.int32)
                    kv = plsc.load_gather(krows[b], [e16, col])
                    qv = plsc.load_gather(qrows[b], [e16, col])
                    acc = acc + kv * qv
                ex = jnp.exp(acc * scale)
                plsc.store_scatter(exb[b], [e16, jnp.full((16,), h, jnp.int32)],
                                   ex)
                for d in range(HD):
                    col = jnp.full((16,), h * HD + d, jnp.int32)
                    vv = plsc.load_gather(vrows[b], [e16, col])
                    plsc.store_scatter(vrows[b], [e16, col], vv * ex)
            return 0

        lax.fori_loop(0, CH // 16, group_body, 0)

    prep_indices(0, 0)
    issue_gathers(0, 0)

    def half(c, b):
        nb = 1 - b
        wait_gathers(c, b)
        compute(b)

        @pl.when(c >= 1)
        def _():
            wait_scatters(nb)

        @pl.when(c + 1 < NCHUNK)
        def _():
            prep_indices(c + 1, nb)
            issue_gathers(c + 1, nb)

        issue_scatters(b)

    def pair_body(i, _):
        half(2 * i, 0)
        half(2 * i + 1, 1)
        return 0

    lax.fori_loop(0, NCHUNK // 2, pair_body, 0)
    wait_scatters(1)
    plsc.subcore_barrier()

    poff = 0
    for w in WBP:
        off = sid * RPW + poff
        pltpu.sync_copy(agg_sh.at[pl.ds(off, w)], stg.at[pl.ds(0, w)])
        pltpu.sync_copy(stg.at[pl.ds(0, w)], agg_out.at[cid, pl.ds(off, w)])
        poff += w
    pltpu.sync_copy(sum_sh.at[pl.ds(sid * RPW, RPW)], stg8)
    pltpu.sync_copy(stg8, sum_out.at[cid, pl.ds(sid * RPW, RPW)])


@functools.cache
def _edge_kernel_fn():
    return functools.partial(
        pl.kernel,
        out_type=[
            jax.ShapeDtypeStruct((NC, NP, CD), jnp.float32),
            jax.ShapeDtypeStruct((NC, NP, H), jnp.float32),
        ],
        mesh=plsc.VectorSubcoreMesh(core_axis_name="c", subcore_axis_name="s",
                                    num_cores=NC, num_subcores=NS),
        compiler_params=pltpu.CompilerParams(needs_layout_passes=False,
                                             use_tc_tiling_on_sc=False),
        scratch_types=[
            pltpu.VMEM((EPS,), jnp.int32),
            pltpu.VMEM((EPS,), jnp.int32),
            [pltpu.VMEM((CH,), jnp.int32)] * 2,
            [pltpu.VMEM((CH,), jnp.int32)] * 2,
            [pltpu.VMEM((CH, CD), jnp.float32)] * 2,
            [pltpu.VMEM((CH, CD), jnp.float32)] * 2,
            [pltpu.VMEM((CH, CD), jnp.float32)] * 2,
            [pltpu.VMEM((CH, H), jnp.float32)] * 2,
            [[pltpu.SemaphoreType.DMA] * 3] * 2,
            [[pltpu.SemaphoreType.DMA] * 2] * 2,
            pltpu.VMEM((WB, CD), jnp.float32),
            pltpu.VMEM((RPW, H), jnp.float32),
            pltpu.VMEM_SHARED((NP, CD), jnp.float32),
            pltpu.VMEM_SHARED((NP, H), jnp.float32),
        ],
    )(_edge_body)


def _epi_body(x_ref, agg_ref, sum_ref, wo_ref, bo_ref, w1a_ref, w1b_ref,
              b1_ref, w2_ref, b2_ref, o_ref):
    ssum = jnp.concatenate([sum_ref[0], sum_ref[1]], axis=1)
    rec = jnp.where(ssum > 0, 1.0 / ssum, 0.0)
    rows = lax.broadcasted_iota(jnp.int32, (2 * H, DIM), 0)
    cols = lax.broadcasted_iota(jnp.int32, (2 * H, DIM), 1)
    hit = (rows // H == cols // CD) & (rows % H == (cols % CD) // HD)
    expand = jnp.where(hit, 1.0, 0.0).astype(jnp.float32)
    recx = jnp.dot(rec, expand, preferred_element_type=jnp.float32,
                   precision=lax.Precision.HIGHEST)
    agg = jnp.concatenate([agg_ref[0], agg_ref[1]], axis=1) * recx
    aggo = jnp.dot(agg, wo_ref[...], preferred_element_type=jnp.float32,
                   precision=lax.Precision.HIGHEST) + bo_ref[...]
    f = (jnp.dot(x_ref[...], w1a_ref[...], preferred_element_type=jnp.float32,
                 precision=lax.Precision.HIGHEST)
         + jnp.dot(aggo, w1b_ref[...], preferred_element_type=jnp.float32,
                   precision=lax.Precision.HIGHEST) + b1_ref[...])
    g = 0.5 * f * (1.0 + lax.erf(f * jnp.float32(1.0 / np.sqrt(2.0))))
    o_ref[...] = jnp.dot(g, w2_ref[...], preferred_element_type=jnp.float32,
                         precision=lax.Precision.HIGHEST) + b2_ref[...]


def _epilogue(x, agg_p, sum_p, W_o, b_o, W1a, W1b, b1, W2, b2):
    grid = N // _BLK
    return pl.pallas_call(
        _epi_body,
        grid=(grid,),
        in_specs=[
            pl.BlockSpec((_BLK, DIM), lambda i: (i, 0)),
            pl.BlockSpec((NC, _BLK, CD), lambda i: (0, i, 0)),
            pl.BlockSpec((NC, _BLK, H), lambda i: (0, i, 0)),
            pl.BlockSpec((DIM, DIM), lambda i: (0, 0)),
            pl.BlockSpec((1, DIM), lambda i: (0, 0)),
            pl.BlockSpec((DIM, DIM), lambda i: (0, 0)),
            pl.BlockSpec((DIM, DIM), lambda i: (0, 0)),
            pl.BlockSpec((1, DIM), lambda i: (0, 0)),
            pl.BlockSpec((DIM, DIM), lambda i: (0, 0)),
            pl.BlockSpec((1, DIM), lambda i: (0, 0)),
        ],
        out_specs=pl.BlockSpec((_BLK, DIM), lambda i: (i, 0)),
        out_shape=jax.ShapeDtypeStruct((N, DIM), jnp.float32),
    )(x, agg_p, sum_p, W_o, b_o, W1a, W1b, b1, W2, b2)


_PERM = np.concatenate([
    np.concatenate([np.arange(h * 3 * HD + j * HD, h * 3 * HD + (j + 1) * HD)
                    for h in range(H)])
    for j in range(3)
])


def kernel(x, edge_index, W_qkv, b_qkv, W_o, b_o, W1, b1, W2, b2):
    w_perm = W_qkv[:, _PERM]
    b_perm = b_qkv[_PERM]
    bq = b_perm[:DIM].reshape(1, DIM)
    bk = b_perm[DIM:2 * DIM].reshape(1, DIM)
    bv = b_perm[2 * DIM:].reshape(1, DIM)

    q, k, v = _qkv_project(x, w_perm, bq, bk, bv)

    src = edge_index[0]
    dst = edge_index[1]
    zagg = jnp.zeros((RPW, CD), jnp.float32)
    zsum = jnp.zeros((RPW, H), jnp.float32)
    agg_p, sum_p = _edge_kernel_fn()(
        q.reshape(NC * N, CD), k.reshape(NC * N, CD), v.reshape(NC * N, CD),
        src, dst, zagg, zsum)

    return _epilogue(x, agg_p, sum_p, W_o, b_o.reshape(1, DIM),
                     W1[:DIM], W1[DIM:], b1.reshape(1, DIM), W2,
                     b2.reshape(1, DIM))


# speedup vs baseline: 13.2651x; 1.0002x over previous
import functools

import jax
import jax.numpy as jnp
import numpy as np
from jax import lax
from jax.experimental import pallas as pl
from jax.experimental.pallas import tpu as pltpu
from jax.experimental.pallas import tpu_sc as plsc

N = 10000
E = 320000
DIM = 128
H = 8
HD = 16

NC = 2
NS = 16
HC = H // NC
CD = HC * HD
EPS = E // NS
CH = 80
NCHUNK = EPS // CH
NP = 10112
RPW = NP // NS
WBP = (128, 128, 128, 128, 120)


_BLK = 2000


def _qkv_body(x_ref, w_ref, bq_ref, bk_ref, bv_ref, q_ref, k_ref, v_ref):
    acc = jnp.dot(x_ref[...], w_ref[...], preferred_element_type=jnp.float32,
                  precision=lax.Precision.HIGHEST)
    q = acc[:, :DIM] + bq_ref[...]
    k = acc[:, DIM:2 * DIM] + bk_ref[...]
    v = acc[:, 2 * DIM:] + bv_ref[...]
    q_ref[0] = q[:, :CD]
    q_ref[1] = q[:, CD:]
    k_ref[0] = k[:, :CD]
    k_ref[1] = k[:, CD:]
    v_ref[0] = v[:, :CD]
    v_ref[1] = v[:, CD:]


def _qkv_project(x, w_perm, bq, bk, bv):
    grid = N // _BLK
    return pl.pallas_call(
        _qkv_body,
        grid=(grid,),
        in_specs=[
            pl.BlockSpec((_BLK, DIM), lambda i: (i, 0)),
            pl.BlockSpec((DIM, 3 * DIM), lambda i: (0, 0)),
            pl.BlockSpec((1, DIM), lambda i: (0, 0)),
            pl.BlockSpec((1, DIM), lambda i: (0, 0)),
            pl.BlockSpec((1, DIM), lambda i: (0, 0)),
        ],
        out_specs=[
            pl.BlockSpec((NC, _BLK, CD), lambda i: (0, i, 0)),
            pl.BlockSpec((NC, _BLK, CD), lambda i: (0, i, 0)),
            pl.BlockSpec((NC, _BLK, CD), lambda i: (0, i, 0)),
        ],
        out_shape=[jax.ShapeDtypeStruct((NC, N, CD), jnp.float32)] * 3,
    )(x, w_perm, bq, bk, bv)


def _edge_body(q_hbm, k_hbm, v_hbm, src_hbm, dst_hbm, zagg_hbm, zsum_hbm,
               agg_out, sum_out, src_all, dst_all, dst_s, dstq_s, krows,
               qrows, vrows, exb, gsem, ssem, stg, stg8, agg_sh, sum_sh):
    cid = lax.axis_index("c")
    sid = lax.axis_index("s")

    poff = 0
    for w in WBP:
        off = sid * RPW + poff
        pltpu.sync_copy(zagg_hbm.at[pl.ds(poff, w)], stg.at[pl.ds(0, w)])
        pltpu.sync_copy(stg.at[pl.ds(0, w)], agg_sh.at[pl.ds(off, w)])
        poff += w
    pltpu.sync_copy(zsum_hbm.at[pl.ds(0, RPW)], stg8)
    pltpu.sync_copy(stg8, sum_sh.at[pl.ds(sid * RPW, RPW)])
    plsc.subcore_barrier()

    lanes = lax.iota(jnp.int32, 16)
    scale = jnp.float32(1.0 / np.sqrt(HD))
    roff = jnp.broadcast_to(cid * N, (16,)).astype(jnp.int32)

    pltpu.sync_copy(src_hbm.at[pl.ds(sid * EPS, EPS)], src_all)
    pltpu.sync_copy(dst_hbm.at[pl.ds(sid * EPS, EPS)], dst_all)

    def adj_body(i, _):
        sl = pl.ds(i * 16, 16)
        src_all[sl] = src_all[sl] + roff
        return 0

    lax.fori_loop(0, EPS // 16, adj_body, 0)

    zero16 = jnp.zeros((16,), jnp.float32)
    for b in range(2):
        for g in range(CH // 16):
            e16 = g * 16 + lanes
            for h in range(HC, H):
                plsc.store_scatter(exb[b], [e16, jnp.full((16,), h, jnp.int32)],
                                   zero16)

    def prep_indices(c, b):
        for g in range(CH // 16):
            sl = pl.ds(g * 16, 16)
            d16 = dst_all[pl.ds(c * CH + g * 16, 16)]
            dst_s[b][sl] = d16
            dstq_s[b][sl] = d16 + roff

    def issue_gathers(c, b):
        base = pl.ds(c * CH, CH)
        pltpu.async_copy(k_hbm.at[src_all.at[base]], krows[b], gsem[b][0])
        pltpu.async_copy(q_hbm.at[dstq_s[b]], qrows[b], gsem[b][1])
        pltpu.async_copy(v_hbm.at[src_all.at[base]], vrows[b], gsem[b][2])

    def wait_gathers(c, b):
        base = pl.ds(c * CH, CH)
        pltpu.make_async_copy(k_hbm.at[src_all.at[base]], krows[b],
                              gsem[b][0]).wait()
        pltpu.make_async_copy(q_hbm.at[dstq_s[b]], qrows[b], gsem[b][1]).wait()
        pltpu.make_async_copy(v_hbm.at[src_all.at[base]], vrows[b],
                              gsem[b][2]).wait()

    def issue_scatters(b):
        pltpu.async_copy(exb[b], sum_sh.at[dst_s[b]], ssem[b][0], add=True)
        pltpu.async_copy(vrows[b], agg_sh.at[dst_s[b]], ssem[b][1], add=True)

    def wait_scatters(b):
        pltpu.make_async_copy(exb[b], sum_sh.at[dst_s[b]], ssem[b][0]).wait()
        pltpu.make_async_copy(vrows[b], agg_sh.at[dst_s[b]], ssem[b][1]).wait()

    def compute(b):
        def group_body(g, _):
            e16 = g * 16 + lanes
            for h in range(HC):
                acc = jnp.zeros((16,), jnp.float32)
                for d in range(HD):
                    col = jnp.full((16,), h * HD + d, jnp.int32)
                    kv = plsc.load_gather(krows[b], [e16, col])
                    qv = plsc.load_gather(qrows[b], [e16, col])
                    acc = acc + kv * qv
                ex = jnp.exp(acc * scale)
                plsc.store_scatter(exb[b], [e16, jnp.full((16,), h, jnp.int32)],
                                   ex)
                for d in range(HD):
                    col = jnp.full((16,), h * HD + d, jnp.int32)
                    vv = plsc.load_gather(vrows[b], [e16, col])
                    plsc.store_scatter(vrows[b], [e16, col], vv * ex)
            return 0

        lax.fori_loop(0, CH // 16, group_body, 0)

    prep_indices(0, 0)
    issue_gathers(0, 0)

    def half(c, b):
        nb = 1 - b
        wait_gathers(c, b)
        compute(b)

        @pl.when(c >= 1)
        def _():
            wait_scatters(nb)

        @pl.when(c + 1 < NCHUNK)
        def _():
            prep_indices(c + 1, nb)
            issue_gathers(c + 1, nb)

        issue_scatters(b)

    def pair_body(i, _):
        half(2 * i, 0)
        half(2 * i + 1, 1)
        return 0

    lax.fori_loop(0, NCHUNK // 2, pair_body, 0)
    wait_scatters(1)
    plsc.subcore_barrier()

    poff = 0
    for w in WBP:
        off = sid * RPW + poff
        pltpu.sync_copy(agg_sh.at[pl.ds(off, w)], stg.at[pl.ds(0, w)])
        pltpu.sync_copy(stg.at[pl.ds(0, w)], agg_out.at[cid, pl.ds(off, w)])
        poff += w
    pltpu.sync_copy(sum_sh.at[pl.ds(sid * RPW, RPW)], stg8)
    pltpu.sync_copy(stg8, sum_out.at[cid, pl.ds(sid * RPW, RPW)])


@functools.cache
def _edge_kernel_fn():
    return functools.partial(
        pl.kernel,
        out_type=[
            jax.ShapeDtypeStruct((NC, NP, CD), jnp.float32),
            jax.ShapeDtypeStruct((NC, NP, H), jnp.float32),
        ],
        mesh=plsc.VectorSubcoreMesh(core_axis_name="c", subcore_axis_name="s",
                                    num_cores=NC, num_subcores=NS),
        compiler_params=pltpu.CompilerParams(needs_layout_passes=False,
                                             use_tc_tiling_on_sc=False),
        scratch_types=[
            pltpu.VMEM((EPS,), jnp.int32),
            pltpu.VMEM((EPS,), jnp.int32),
            [pltpu.VMEM((CH,), jnp.int32)] * 2,
            [pltpu.VMEM((CH,), jnp.int32)] * 2,
            [pltpu.VMEM((CH, CD), jnp.float32)] * 2,
            [pltpu.VMEM((CH, CD), jnp.float32)] * 2,
            [pltpu.VMEM((CH, CD), jnp.float32)] * 2,
            [pltpu.VMEM((CH, H), jnp.float32)] * 2,
            [[pltpu.SemaphoreType.DMA] * 3] * 2,
            [[pltpu.SemaphoreType.DMA] * 2] * 2,
            pltpu.VMEM((128, CD), jnp.float32),
            pltpu.VMEM((RPW, H), jnp.float32),
            pltpu.VMEM_SHARED((NP, CD), jnp.float32),
            pltpu.VMEM_SHARED((NP, H), jnp.float32),
        ],
    )(_edge_body)


def _epi_body(x_ref, agg_ref, sum_ref, wo_ref, bo_ref, w1a_ref, w1b_ref,
              b1_ref, w2_ref, b2_ref, o_ref):
    ssum = jnp.concatenate([sum_ref[0], sum_ref[1]], axis=1)
    rec = jnp.where(ssum > 0, 1.0 / ssum, 0.0)
    rows = lax.broadcasted_iota(jnp.int32, (2 * H, DIM), 0)
    cols = lax.broadcasted_iota(jnp.int32, (2 * H, DIM), 1)
    hit = (rows // H == cols // CD) & (rows % H == (cols % CD) // HD)
    expand = jnp.where(hit, 1.0, 0.0).astype(jnp.float32)
    recx = jnp.dot(rec, expand, preferred_element_type=jnp.float32,
                   precision=lax.Precision.HIGHEST)
    agg = jnp.concatenate([agg_ref[0], agg_ref[1]], axis=1) * recx
    aggo = jnp.dot(agg, wo_ref[...], preferred_element_type=jnp.float32,
                   precision=lax.Precision.HIGHEST) + bo_ref[...]
    f = (jnp.dot(x_ref[...], w1a_ref[...], preferred_element_type=jnp.float32,
                 precision=lax.Precision.HIGHEST)
         + jnp.dot(aggo, w1b_ref[...], preferred_element_type=jnp.float32,
                   precision=lax.Precision.HIGHEST) + b1_ref[...])
    g = 0.5 * f * (1.0 + lax.erf(f * jnp.float32(1.0 / np.sqrt(2.0))))
    o_ref[...] = jnp.dot(g, w2_ref[...], preferred_element_type=jnp.float32,
                         precision=lax.Precision.HIGHEST) + b2_ref[...]


def _epilogue(x, agg_p, sum_p, W_o, b_o, W1a, W1b, b1, W2, b2):
    grid = N // _BLK
    return pl.pallas_call(
        _epi_body,
        grid=(grid,),
        in_specs=[
            pl.BlockSpec((_BLK, DIM), lambda i: (i, 0)),
            pl.BlockSpec((NC, _BLK, CD), lambda i: (0, i, 0)),
            pl.BlockSpec((NC, _BLK, H), lambda i: (0, i, 0)),
            pl.BlockSpec((DIM, DIM), lambda i: (0, 0)),
            pl.BlockSpec((1, DIM), lambda i: (0, 0)),
            pl.BlockSpec((DIM, DIM), lambda i: (0, 0)),
            pl.BlockSpec((DIM, DIM), lambda i: (0, 0)),
            pl.BlockSpec((1, DIM), lambda i: (0, 0)),
            pl.BlockSpec((DIM, DIM), lambda i: (0, 0)),
            pl.BlockSpec((1, DIM), lambda i: (0, 0)),
        ],
        out_specs=pl.BlockSpec((_BLK, DIM), lambda i: (i, 0)),
        out_shape=jax.ShapeDtypeStruct((N, DIM), jnp.float32),
    )(x, agg_p, sum_p, W_o, b_o, W1a, W1b, b1, W2, b2)


_PERM = np.concatenate([
    np.concatenate([np.arange(h * 3 * HD + j * HD, h * 3 * HD + (j + 1) * HD)
                    for h in range(H)])
    for j in range(3)
])


def kernel(x, edge_index, W_qkv, b_qkv, W_o, b_o, W1, b1, W2, b2):
    w_perm = W_qkv[:, _PERM]
    b_perm = b_qkv[_PERM]
    bq = b_perm[:DIM].reshape(1, DIM)
    bk = b_perm[DIM:2 * DIM].reshape(1, DIM)
    bv = b_perm[2 * DIM:].reshape(1, DIM)

    q, k, v = _qkv_project(x, w_perm, bq, bk, bv)

    src = edge_index[0]
    dst = edge_index[1]
    zagg = jnp.zeros((RPW, CD), jnp.float32)
    zsum = jnp.zeros((RPW, H), jnp.float32)
    agg_p, sum_p = _edge_kernel_fn()(
        q.reshape(NC * N, CD), k.reshape(NC * N, CD), v.reshape(NC * N, CD),
        src, dst, zagg, zsum)

    return _epilogue(x, agg_p, sum_p, W_o, b_o.reshape(1, DIM),
                     W1[:DIM], W1[DIM:], b1.reshape(1, DIM), W2,
                     b2.reshape(1, DIM))
